# baseline (speedup 1.0000x reference)
import functools
import math

import jax
import jax.numpy as jnp
from jax.experimental import pallas as pl

IMG_SIZE = 1024.0
EMBED_DIM = 256
NUM_POS_FEATS = 128
B, N = 4096, 50
BN = B * N

TWO_PI = 2.0 * math.pi


def _body(c_ref, l_ref, g_ref, t_ref, o_ref):
    c = c_ref[...]
    g = g_ref[...]
    lab = l_ref[...]
    t = t_ref[...]

    cn = (c + 0.5) * (2.0 / IMG_SIZE) - 1.0
    cnb = cn.astype(jnp.bfloat16).astype(jnp.float32)
    gb = g.astype(jnp.bfloat16).astype(jnp.float32)
    phase = (cnb[:, 0:1] * gb[0:1, :] + cnb[:, 1:2] * gb[1:2, :]) * TWO_PI
    pe = jnp.concatenate([jnp.sin(phase), jnp.cos(phase)], axis=-1)

    keep = (lab != -1).astype(jnp.float32)
    acc = keep * pe
    for k in range(5):
        w = (lab == (k - 1)).astype(jnp.float32)
        acc = acc + w * t[k:k + 1, :]
    o_ref[...] = acc


@functools.partial(jax.jit, static_argnames=())
def kernel(point_coords, point_labels, gaussian_matrix, not_a_point_embed,
           pe0, pe1, pe2, pe3):
    coords = point_coords.reshape(BN, 2)
    labels = point_labels.reshape(BN, 1).astype(jnp.int32)
    table = jnp.stack([not_a_point_embed, pe0, pe1, pe2, pe3])

    R = 2048
    out = pl.pallas_call(
        _body,
        grid=(BN // R,),
        in_specs=[
            pl.BlockSpec((R, 2), lambda i: (i, 0)),
            pl.BlockSpec((R, 1), lambda i: (i, 0)),
            pl.BlockSpec((2, NUM_POS_FEATS), lambda i: (0, 0)),
            pl.BlockSpec((5, EMBED_DIM), lambda i: (0, 0)),
        ],
        out_specs=pl.BlockSpec((R, EMBED_DIM), lambda i: (i, 0)),
        out_shape=jax.ShapeDtypeStruct((BN, EMBED_DIM), jnp.float32),
    )(coords, labels, gaussian_matrix, table)
    return out.reshape(B, N, EMBED_DIM)


# speedup vs baseline: 1.3730x; 1.3730x over previous
import functools
import math

import jax
import jax.numpy as jnp
from jax.experimental import pallas as pl

IMG_SIZE = 1024.0
EMBED_DIM = 256
NUM_POS_FEATS = 128
B, N = 4096, 50
BN = B * N

TWO_PI = 2.0 * math.pi

_MAGIC = 12582912.0

_SIN_C = (6.28318282, -41.34142155, 81.59618767, -76.58013845,
          41.20556003, -12.27152265)
_COS_C = (0.99999999, -19.73920453, 64.93911913, -85.45016824,
          60.16785437, -25.96840047, 6.52973539)


def _horner(coefs, x):
    acc = jnp.full_like(x, coefs[-1])
    for c in coefs[-2::-1]:
        acc = acc * x + c
    return acc


def _body(c_ref, l_ref, g_ref, t_ref, o_ref):
    c = c_ref[...]
    g = g_ref[...]
    lab = l_ref[...]
    t = t_ref[...]

    cn = (c + 0.5) * (2.0 / IMG_SIZE) - 1.0
    cnb = cn.astype(jnp.bfloat16).astype(jnp.float32)
    gb = g.astype(jnp.bfloat16).astype(jnp.float32)
    turns = cnb[:, 0:1] * gb[0:1, :] + cnb[:, 1:2] * gb[1:2, :]
    q = turns - jnp.round(turns)
    s2 = q * q
    sin_v = _horner(_SIN_C, s2) * q
    cos_v = _horner(_COS_C, s2)
    pe = jnp.concatenate([sin_v, cos_v], axis=-1)

    keep = (lab != -1).astype(jnp.float32)
    acc = keep * pe
    for k in range(5):
        w = (lab == (k - 1)).astype(jnp.float32)
        acc = acc + w * t[k:k + 1, :]
    o_ref[...] = acc


@functools.partial(jax.jit, static_argnames=())
def kernel(point_coords, point_labels, gaussian_matrix, not_a_point_embed,
           pe0, pe1, pe2, pe3):
    coords = point_coords.reshape(BN, 2)
    labels = point_labels.reshape(BN, 1).astype(jnp.int32)
    table = jnp.stack([not_a_point_embed, pe0, pe1, pe2, pe3])

    R = 2048
    out = pl.pallas_call(
        _body,
        grid=(BN // R,),
        in_specs=[
            pl.BlockSpec((R, 2), lambda i: (i, 0)),
            pl.BlockSpec((R, 1), lambda i: (i, 0)),
            pl.BlockSpec((2, NUM_POS_FEATS), lambda i: (0, 0)),
            pl.BlockSpec((5, EMBED_DIM), lambda i: (0, 0)),
        ],
        out_specs=pl.BlockSpec((R, EMBED_DIM), lambda i: (i, 0)),
        out_shape=jax.ShapeDtypeStruct((BN, EMBED_DIM), jnp.float32),
    )(coords, labels, gaussian_matrix, table)
    return out.reshape(B, N, EMBED_DIM)


# speedup vs baseline: 1.5409x; 1.1223x over previous
import functools
import math

import jax
import jax.numpy as jnp
from jax.experimental import pallas as pl

IMG_SIZE = 1024.0
EMBED_DIM = 256
NUM_POS_FEATS = 128
B, N = 4096, 50
BN = B * N

_SIN_C = (6.28318282, -41.34142155, 81.59618767, -76.58013845,
          41.20556003, -12.27152265)
_COS_C = (0.99999999, -19.73920453, 64.93911913, -85.45016824,
          60.16785437, -25.96840047, 6.52973539)


def _horner(coefs, x):
    acc = jnp.full_like(x, coefs[-1])
    for c in coefs[-2::-1]:
        acc = acc * x + c
    return acc


def _body(c_ref, l_ref, g_ref, w_ref, o_ref):
    c = c_ref[...]
    g = g_ref[...]
    lab = l_ref[...]
    w = w_ref[...]

    cn = (c + 0.5) * (2.0 / IMG_SIZE) - 1.0
    turns = jnp.dot(cn.astype(jnp.bfloat16), g.astype(jnp.bfloat16),
                    preferred_element_type=jnp.float32)
    q = turns - jnp.round(turns)
    s2 = q * q
    sin_v = _horner(_SIN_C, s2) * q
    cos_v = _horner(_COS_C, s2)

    iota = jax.lax.broadcasted_iota(jnp.int32, (lab.shape[0], 8), 1) - 1
    oh = (lab == iota).astype(jnp.bfloat16)
    m = jnp.dot(oh, w.astype(jnp.bfloat16),
                preferred_element_type=jnp.float32)
    keep = 1.0 - m[:, 2 * NUM_POS_FEATS:]
    o_ref[:, :NUM_POS_FEATS] = sin_v * keep + m[:, :NUM_POS_FEATS]
    o_ref[:, NUM_POS_FEATS:] = cos_v * keep + m[:, NUM_POS_FEATS:2 * NUM_POS_FEATS]


@functools.partial(jax.jit, static_argnames=())
def kernel(point_coords, point_labels, gaussian_matrix, not_a_point_embed,
           pe0, pe1, pe2, pe3):
    coords = point_coords.reshape(BN, 2)
    labels = point_labels.reshape(BN, 1).astype(jnp.int32)
    table = jnp.stack([not_a_point_embed, pe0, pe1, pe2, pe3])
    table = jnp.concatenate(
        [table, jnp.zeros((3, EMBED_DIM), jnp.float32)], axis=0)
    kill = jnp.zeros((8, NUM_POS_FEATS), jnp.float32).at[0, :].set(1.0)
    wmat = jnp.concatenate([table, kill], axis=1)

    R = 2048
    out = pl.pallas_call(
        _body,
        grid=(BN // R,),
        in_specs=[
            pl.BlockSpec((R, 2), lambda i: (i, 0)),
            pl.BlockSpec((R, 1), lambda i: (i, 0)),
            pl.BlockSpec((2, NUM_POS_FEATS), lambda i: (0, 0)),
            pl.BlockSpec((8, 3 * NUM_POS_FEATS), lambda i: (0, 0)),
        ],
        out_specs=pl.BlockSpec((R, EMBED_DIM), lambda i: (i, 0)),
        out_shape=jax.ShapeDtypeStruct((BN, EMBED_DIM), jnp.float32),
    )(coords, labels, gaussian_matrix, wmat)
    return out.reshape(B, N, EMBED_DIM)


# speedup vs baseline: 5.5814x; 3.6221x over previous
import functools

import jax
import jax.numpy as jnp
from jax.experimental import pallas as pl

IMG_SIZE = 1024.0
EMBED_DIM = 256
NUM_POS_FEATS = 128
B, N = 4096, 50
BN = B * N

_SIN_C = (6.28318282, -41.34142155, 81.59618767, -76.58013845,
          41.20556003, -12.27152265)
_COS_C = (0.99999999, -19.73920453, 64.93911913, -85.45016824,
          60.16785437, -25.96840047, 6.52973539)


def _horner(coefs, x):
    acc = jnp.full_like(x, coefs[-1])
    for c in coefs[-2::-1]:
        acc = acc * x + c
    return acc


def _body(cx_ref, cy_ref, l_ref, wg_ref, wk_ref, ws_ref, wc_ref, o_ref):
    cx = cx_ref[...]
    cy = cy_ref[...]
    lab = l_ref[...]

    cnx = ((cx + 0.5) * (2.0 / IMG_SIZE) - 1.0).astype(jnp.bfloat16)
    cny = ((cy + 0.5) * (2.0 / IMG_SIZE) - 1.0).astype(jnp.bfloat16)
    x2 = jnp.concatenate([cnx, cny], axis=1)
    turns = jnp.dot(x2, wg_ref[...],
                    preferred_element_type=jnp.float32)
    q = turns - jnp.round(turns)
    s2 = q * q
    sin_v = _horner(_SIN_C, s2) * q
    cos_v = _horner(_COS_C, s2)

    oh5 = jnp.concatenate([(lab == k).astype(jnp.bfloat16)
                           for k in (-1, 0, 1, 2, 3)], axis=1)
    kill = jnp.dot(oh5[:, :N], wk_ref[...],
                   preferred_element_type=jnp.float32)
    csin = jnp.dot(oh5, ws_ref[...],
                   preferred_element_type=jnp.float32)
    ccos = jnp.dot(oh5, wc_ref[...],
                   preferred_element_type=jnp.float32)

    sin_m = sin_v - sin_v * kill + csin
    cos_m = cos_v - cos_v * kill + ccos
    for n in range(N):
        sl = slice(n * NUM_POS_FEATS, (n + 1) * NUM_POS_FEATS)
        o_ref[n, :, :NUM_POS_FEATS] = sin_m[:, sl]
        o_ref[n, :, NUM_POS_FEATS:] = cos_m[:, sl]


@functools.partial(jax.jit, static_argnames=())
def kernel(point_coords, point_labels, gaussian_matrix, not_a_point_embed,
           pe0, pe1, pe2, pe3):
    cx = point_coords[:, :, 0]
    cy = point_coords[:, :, 1]
    labels = point_labels.astype(jnp.int32)

    eye = jnp.eye(N, dtype=jnp.float32)
    gb = gaussian_matrix.astype(jnp.bfloat16).astype(jnp.float32)
    wg = jnp.concatenate([jnp.kron(eye, gb[0:1, :]),
                          jnp.kron(eye, gb[1:2, :])], axis=0)
    wk = jnp.kron(eye, jnp.ones((1, NUM_POS_FEATS), jnp.float32))
    table = jnp.stack([not_a_point_embed, pe0, pe1, pe2, pe3])
    ws = jnp.kron(table[:, None, :NUM_POS_FEATS], eye[:, :, None]
                  ).reshape(5 * N, N * NUM_POS_FEATS)
    wc = jnp.kron(table[:, None, NUM_POS_FEATS:], eye[:, :, None]
                  ).reshape(5 * N, N * NUM_POS_FEATS)

    bb = 128
    out = pl.pallas_call(
        _body,
        grid=(B // bb,),
        in_specs=[
            pl.BlockSpec((bb, N), lambda i: (i, 0)),
            pl.BlockSpec((bb, N), lambda i: (i, 0)),
            pl.BlockSpec((bb, N), lambda i: (i, 0)),
            pl.BlockSpec((2 * N, N * NUM_POS_FEATS), lambda i: (0, 0)),
            pl.BlockSpec((N, N * NUM_POS_FEATS), lambda i: (0, 0)),
            pl.BlockSpec((5 * N, N * NUM_POS_FEATS), lambda i: (0, 0)),
            pl.BlockSpec((5 * N, N * NUM_POS_FEATS), lambda i: (0, 0)),
        ],
        out_specs=pl.BlockSpec((N, bb, EMBED_DIM), lambda i: (0, i, 0)),
        out_shape=jax.ShapeDtypeStruct((N, B, EMBED_DIM), jnp.float32),
    )(cx, cy, labels, wg.astype(jnp.bfloat16), wk.astype(jnp.bfloat16),
      ws.astype(jnp.bfloat16), wc.astype(jnp.bfloat16))
    return out.transpose(1, 0, 2)


# speedup vs baseline: 6.2917x; 1.1273x over previous
import functools

import jax
import jax.numpy as jnp
from jax.experimental import pallas as pl

IMG_SIZE = 1024.0
EMBED_DIM = 256
NUM_POS_FEATS = 128
B, N = 4096, 50
BN = B * N

_SIN_C = (6.283161527975795, -41.33688334283728, 81.44874586130172,
          -74.9161270501732, 33.56187608886677)
_COS_C = (0.9999598186038352, -19.73104541952081, 64.67351931628222,
          -82.40420032794171, 45.64802504853961)


def _horner(coefs, x):
    acc = jnp.full_like(x, coefs[-1])
    for c in coefs[-2::-1]:
        acc = acc * x + c
    return acc


def _body(cx_ref, cy_ref, l_ref, wg_ref, wk_ref, ws_ref, wc_ref, o_ref):
    cx = cx_ref[...]
    cy = cy_ref[...]
    lab = l_ref[...]

    cnx = ((cx + 0.5) * (2.0 / IMG_SIZE) - 1.0).astype(jnp.bfloat16)
    cny = ((cy + 0.5) * (2.0 / IMG_SIZE) - 1.0).astype(jnp.bfloat16)
    x2 = jnp.concatenate([cnx, cny], axis=1)
    turns = jnp.dot(x2, wg_ref[...],
                    preferred_element_type=jnp.float32)

    oh5 = jnp.concatenate([(lab == k).astype(jnp.bfloat16)
                           for k in (-1, 0, 1, 2, 3)], axis=1)
    keep = jnp.dot((lab != -1).astype(jnp.bfloat16), wk_ref[...],
                   preferred_element_type=jnp.float32)
    csin = jnp.dot(oh5, ws_ref[...],
                   preferred_element_type=jnp.float32)
    ccos = jnp.dot(oh5, wc_ref[...],
                   preferred_element_type=jnp.float32)

    q = turns - jnp.round(turns)
    q = q * keep
    s2 = q * q
    sin_m = _horner(_SIN_C, s2) * q + csin
    cos_m = _horner(_COS_C, s2) + ccos
    for n in range(N):
        sl = slice(n * NUM_POS_FEATS, (n + 1) * NUM_POS_FEATS)
        o_ref[n, :, :NUM_POS_FEATS] = sin_m[:, sl]
        o_ref[n, :, NUM_POS_FEATS:] = cos_m[:, sl]


@functools.partial(jax.jit, static_argnames=())
def kernel(point_coords, point_labels, gaussian_matrix, not_a_point_embed,
           pe0, pe1, pe2, pe3):
    cx = point_coords[:, :, 0]
    cy = point_coords[:, :, 1]
    labels = point_labels.astype(jnp.int32)

    eye = jnp.eye(N, dtype=jnp.float32)
    gb = gaussian_matrix.astype(jnp.bfloat16).astype(jnp.float32)
    wg = jnp.concatenate([jnp.kron(eye, gb[0:1, :]),
                          jnp.kron(eye, gb[1:2, :])], axis=0)
    wk = jnp.kron(eye, jnp.ones((1, NUM_POS_FEATS), jnp.float32))
    table = jnp.stack([not_a_point_embed, pe0, pe1, pe2, pe3])
    table = table.at[0, NUM_POS_FEATS:].add(-_COS_C[0])
    ws = jnp.kron(table[:, None, :NUM_POS_FEATS], eye[:, :, None]
                  ).reshape(5 * N, N * NUM_POS_FEATS)
    wc = jnp.kron(table[:, None, NUM_POS_FEATS:], eye[:, :, None]
                  ).reshape(5 * N, N * NUM_POS_FEATS)

    bb = 128
    out = pl.pallas_call(
        _body,
        grid=(B // bb,),
        in_specs=[
            pl.BlockSpec((bb, N), lambda i: (i, 0)),
            pl.BlockSpec((bb, N), lambda i: (i, 0)),
            pl.BlockSpec((bb, N), lambda i: (i, 0)),
            pl.BlockSpec((2 * N, N * NUM_POS_FEATS), lambda i: (0, 0)),
            pl.BlockSpec((N, N * NUM_POS_FEATS), lambda i: (0, 0)),
            pl.BlockSpec((5 * N, N * NUM_POS_FEATS), lambda i: (0, 0)),
            pl.BlockSpec((5 * N, N * NUM_POS_FEATS), lambda i: (0, 0)),
        ],
        out_specs=pl.BlockSpec((N, bb, EMBED_DIM), lambda i: (0, i, 0)),
        out_shape=jax.ShapeDtypeStruct((N, B, EMBED_DIM), jnp.float32),
    )(cx, cy, labels, wg.astype(jnp.bfloat16), wk.astype(jnp.bfloat16),
      ws.astype(jnp.bfloat16), wc.astype(jnp.bfloat16))
    return out.transpose(1, 0, 2)


# speedup vs baseline: 6.6698x; 1.0601x over previous
import functools

import jax
import jax.numpy as jnp
from jax.experimental import pallas as pl

IMG_SIZE = 1024.0
EMBED_DIM = 256
NUM_POS_FEATS = 128
B, N = 4096, 50
BN = B * N

_SIN_C = (6.283161527975795, -41.33688334283728, 81.44874586130172,
          -74.9161270501732, 33.56187608886677)
_COS_C = (0.9999598186038352, -19.73104541952081, 64.67351931628222,
          -82.40420032794171, 45.64802504853961)


def _horner(coefs, x):
    acc = jnp.full_like(x, coefs[-1])
    for c in coefs[-2::-1]:
        acc = acc * x + c
    return acc


def _body(cx_ref, cy_ref, l_ref, wg_ref, wk_ref, ws_ref, wc_ref, o_ref):
    cx = cx_ref[...]
    cy = cy_ref[...]
    lab = l_ref[...]

    cnx = ((cx + 0.5) * (2.0 / IMG_SIZE) - 1.0).astype(jnp.bfloat16)
    cny = ((cy + 0.5) * (2.0 / IMG_SIZE) - 1.0).astype(jnp.bfloat16)
    x2 = jnp.concatenate([cnx, cny], axis=1)
    turns = jnp.dot(x2, wg_ref[...],
                    preferred_element_type=jnp.float32)

    oh5 = jnp.concatenate([(lab == k).astype(jnp.bfloat16)
                           for k in (-1, 0, 1, 2, 3)], axis=1)
    keep = jnp.dot((lab != -1).astype(jnp.bfloat16), wk_ref[...],
                   preferred_element_type=jnp.float32)
    csin = jnp.dot(oh5, ws_ref[...],
                   preferred_element_type=jnp.float32)
    ccos = jnp.dot(oh5, wc_ref[...],
                   preferred_element_type=jnp.float32)

    q = turns - jnp.round(turns)
    q = q * keep
    s2 = q * q
    sin_m = _horner(_SIN_C, s2) * q + csin
    cos_m = _horner(_COS_C, s2) + ccos
    for n in range(N):
        sl = slice(n * NUM_POS_FEATS, (n + 1) * NUM_POS_FEATS)
        o_ref[n, :, :NUM_POS_FEATS] = sin_m[:, sl]
        o_ref[n, :, NUM_POS_FEATS:] = cos_m[:, sl]


@functools.partial(jax.jit, static_argnames=())
def kernel(point_coords, point_labels, gaussian_matrix, not_a_point_embed,
           pe0, pe1, pe2, pe3):
    cx = point_coords[:, :, 0]
    cy = point_coords[:, :, 1]
    labels = point_labels.astype(jnp.int32)

    eye = jnp.eye(N, dtype=jnp.float32)
    gb = gaussian_matrix.astype(jnp.bfloat16).astype(jnp.float32)
    wg = jnp.concatenate([jnp.kron(eye, gb[0:1, :]),
                          jnp.kron(eye, gb[1:2, :])], axis=0)
    wk = jnp.kron(eye, jnp.ones((1, NUM_POS_FEATS), jnp.float32))
    table = jnp.stack([not_a_point_embed, pe0, pe1, pe2, pe3])
    table = table.at[0, NUM_POS_FEATS:].add(-_COS_C[0])
    ws = jnp.kron(table[:, None, :NUM_POS_FEATS], eye[:, :, None]
                  ).reshape(5 * N, N * NUM_POS_FEATS)
    wc = jnp.kron(table[:, None, NUM_POS_FEATS:], eye[:, :, None]
                  ).reshape(5 * N, N * NUM_POS_FEATS)

    bb = 256
    out = pl.pallas_call(
        _body,
        grid=(B // bb,),
        in_specs=[
            pl.BlockSpec((bb, N), lambda i: (i, 0)),
            pl.BlockSpec((bb, N), lambda i: (i, 0)),
            pl.BlockSpec((bb, N), lambda i: (i, 0)),
            pl.BlockSpec((2 * N, N * NUM_POS_FEATS), lambda i: (0, 0)),
            pl.BlockSpec((N, N * NUM_POS_FEATS), lambda i: (0, 0)),
            pl.BlockSpec((5 * N, N * NUM_POS_FEATS), lambda i: (0, 0)),
            pl.BlockSpec((5 * N, N * NUM_POS_FEATS), lambda i: (0, 0)),
        ],
        out_specs=pl.BlockSpec((N, bb, EMBED_DIM), lambda i: (0, i, 0)),
        out_shape=jax.ShapeDtypeStruct((N, B, EMBED_DIM), jnp.float32),
    )(cx, cy, labels, wg.astype(jnp.bfloat16), wk.astype(jnp.bfloat16),
      ws.astype(jnp.bfloat16), wc.astype(jnp.bfloat16))
    return out.transpose(1, 0, 2)


# speedup vs baseline: 7.5485x; 1.1318x over previous
import functools

import jax
import jax.numpy as jnp
from jax.experimental import pallas as pl

IMG_SIZE = 1024.0
EMBED_DIM = 256
NUM_POS_FEATS = 128
B, N = 4096, 50
BN = B * N

_SIN_C = (6.283161527975795, -41.33688334283728, 81.44874586130172,
          -74.9161270501732, 33.56187608886677)
_COS_C = (0.9999598186038352, -19.73104541952081, 64.67351931628222,
          -82.40420032794171, 45.64802504853961)


def _horner(coefs, x):
    acc = jnp.full_like(x, coefs[-1])
    for c in coefs[-2::-1]:
        acc = acc * x + c
    return acc


def _body(cx_ref, cy_ref, l_ref, wg_ref, ws_ref, wc_ref, o_ref):
    cx = cx_ref[...]
    cy = cy_ref[...]
    lab = l_ref[...]

    keepm = (lab != -1).astype(jnp.bfloat16)
    cnx = ((cx + 0.5) * (2.0 / IMG_SIZE) - 1.0).astype(jnp.bfloat16) * keepm
    cny = ((cy + 0.5) * (2.0 / IMG_SIZE) - 1.0).astype(jnp.bfloat16) * keepm
    x2 = jnp.concatenate([cnx, cny], axis=1)
    turns = jnp.dot(x2, wg_ref[...],
                    preferred_element_type=jnp.float32)

    oh5 = jnp.concatenate([(lab == k).astype(jnp.bfloat16)
                           for k in (-1, 0, 1, 2, 3)], axis=1)
    csin = jnp.dot(oh5, ws_ref[...],
                   preferred_element_type=jnp.float32)
    ccos = jnp.dot(oh5, wc_ref[...],
                   preferred_element_type=jnp.float32)

    q = turns - jnp.round(turns)
    s2 = q * q
    sin_m = _horner(_SIN_C, s2) * q + csin
    cos_m = _horner(_COS_C, s2) + ccos
    for n in range(N):
        sl = slice(n * NUM_POS_FEATS, (n + 1) * NUM_POS_FEATS)
        o_ref[n, :, :NUM_POS_FEATS] = sin_m[:, sl]
        o_ref[n, :, NUM_POS_FEATS:] = cos_m[:, sl]


@functools.partial(jax.jit, static_argnames=())
def kernel(point_coords, point_labels, gaussian_matrix, not_a_point_embed,
           pe0, pe1, pe2, pe3):
    cx = point_coords[:, :, 0]
    cy = point_coords[:, :, 1]
    labels = point_labels.astype(jnp.int32)

    eye = jnp.eye(N, dtype=jnp.float32)
    gb = gaussian_matrix.astype(jnp.bfloat16).astype(jnp.float32)
    wg = jnp.concatenate([jnp.kron(eye, gb[0:1, :]),
                          jnp.kron(eye, gb[1:2, :])], axis=0)
    table = jnp.stack([not_a_point_embed, pe0, pe1, pe2, pe3])
    table = table.at[0, NUM_POS_FEATS:].add(-_COS_C[0])
    ws = jnp.kron(table[:, None, :NUM_POS_FEATS], eye[:, :, None]
                  ).reshape(5 * N, N * NUM_POS_FEATS)
    wc = jnp.kron(table[:, None, NUM_POS_FEATS:], eye[:, :, None]
                  ).reshape(5 * N, N * NUM_POS_FEATS)

    bb = 256
    out = pl.pallas_call(
        _body,
        grid=(B // bb,),
        in_specs=[
            pl.BlockSpec((bb, N), lambda i: (i, 0)),
            pl.BlockSpec((bb, N), lambda i: (i, 0)),
            pl.BlockSpec((bb, N), lambda i: (i, 0)),
            pl.BlockSpec((2 * N, N * NUM_POS_FEATS), lambda i: (0, 0)),
            pl.BlockSpec((5 * N, N * NUM_POS_FEATS), lambda i: (0, 0)),
            pl.BlockSpec((5 * N, N * NUM_POS_FEATS), lambda i: (0, 0)),
        ],
        out_specs=pl.BlockSpec((N, bb, EMBED_DIM), lambda i: (0, i, 0)),
        out_shape=jax.ShapeDtypeStruct((N, B, EMBED_DIM), jnp.float32),
    )(cx, cy, labels, wg.astype(jnp.bfloat16),
      ws.astype(jnp.bfloat16), wc.astype(jnp.bfloat16))
    return out.transpose(1, 0, 2)
